# Initial kernel scaffold; baseline (speedup 1.0000x reference)
import jax
import jax.numpy as jnp
from jax import lax
from jax.experimental import pallas as pl
from jax.experimental.pallas import tpu as pltpu
from jax.experimental.pallas import tpu_sc as plsc

NC = 2
NS = 16
LANES = 16
NW = NC * NS
C = 128
ZROWS = 64


def _sc_segment_sum(table, src2d, dst2d, npad, with_cnt):
  n_chunks = src2d.shape[0]
  k = n_chunks // NW
  rows = npad // NS
  nz = rows // ZROWS

  out_type = [jax.ShapeDtypeStruct((NC, npad, 128), jnp.float32)]
  scratch = [
      pltpu.VMEM((k, C), jnp.int32),
      pltpu.VMEM((k, C), jnp.int32),
      pltpu.VMEM((C, 128), jnp.float32),
      pltpu.VMEM((C, 128), jnp.float32),
      pltpu.VMEM((ZROWS, 128), jnp.float32),
      pltpu.VMEM_SHARED((npad, 128), jnp.float32),
      pltpu.SemaphoreType.DMA,
      pltpu.SemaphoreType.DMA,
  ]
  if with_cnt:
    out_type.append(jax.ShapeDtypeStruct((NC, npad, LANES), jnp.float32))
    scratch += [
        pltpu.VMEM((C, LANES), jnp.float32),
        pltpu.VMEM((ZROWS, LANES), jnp.float32),
        pltpu.VMEM_SHARED((npad, LANES), jnp.float32),
    ]

  mesh = plsc.VectorSubcoreMesh(
      core_axis_name="c", subcore_axis_name="s",
      num_cores=NC, num_subcores=NS)

  def body(table_h, src_h, dst_h, agg_h, *rest):
    if with_cnt:
      (cnt_h, idx_s, idx_d, buf0, buf1, zb, acc, sem0, sem1,
       ones, zb16, cacc) = rest
    else:
      (idx_s, idx_d, buf0, buf1, zb, acc, sem0, sem1) = rest
      ones = zb16 = cacc = cnt_h = None

    cid = lax.axis_index("c")
    sid = lax.axis_index("s")
    wid = cid * NS + sid

    pltpu.sync_copy(src_h.at[pl.ds(wid * k, k)], idx_s)
    pltpu.sync_copy(dst_h.at[pl.ds(wid * k, k)], idx_d)

    zeros16 = jnp.zeros((LANES,), jnp.float32)

    def zrow(r, _):
      for cc in range(128 // LANES):
        zb[r, pl.ds(cc * LANES, LANES)] = zeros16
      if with_cnt:
        zb16[r, :] = zeros16
      return 0
    lax.fori_loop(0, ZROWS, zrow, 0)

    if with_cnt:
      lane = lax.iota(jnp.int32, LANES)
      one_row = jnp.where(lane == 0, 1.0, 0.0).astype(jnp.float32)

      def orow(r, _):
        ones[r, :] = one_row
        return 0
      lax.fori_loop(0, C, orow, 0)

    base = sid * rows

    def zacc(i, _):
      pltpu.sync_copy(zb, acc.at[pl.ds(base + i * ZROWS, ZROWS)])
      if with_cnt:
        pltpu.sync_copy(zb16, cacc.at[pl.ds(base + i * ZROWS, ZROWS)])
      return 0
    lax.fori_loop(0, nz, zacc, 0)

    plsc.subcore_barrier()

    pltpu.async_copy(table_h.at[idx_s.at[0]], buf0, sem0)
    pltpu.async_copy(table_h.at[idx_s.at[1]], buf1, sem1)

    def do_chunk(j, buf, sem):
      pltpu.make_async_copy(table_h.at[idx_s.at[j]], buf, sem).wait()
      pltpu.sync_copy(buf, acc.at[idx_d.at[j]], add=True)
      if with_cnt:
        pltpu.sync_copy(ones, cacc.at[idx_d.at[j]], add=True)

      @pl.when(j + 2 < k)
      def _():
        pltpu.async_copy(table_h.at[idx_s.at[j + 2]], buf, sem)

    def loop_body(i2, _):
      do_chunk(i2 * 2, buf0, sem0)
      do_chunk(i2 * 2 + 1, buf1, sem1)
      return 0
    lax.fori_loop(0, k // 2, loop_body, 0)

    plsc.subcore_barrier()

    pltpu.sync_copy(acc.at[pl.ds(base, rows)],
                    agg_h.at[cid, pl.ds(base, rows)])
    if with_cnt:
      pltpu.sync_copy(cacc.at[pl.ds(base, rows)],
                      cnt_h.at[cid, pl.ds(base, rows)])

  f = pl.kernel(body, out_type=tuple(out_type), mesh=mesh,
                scratch_types=tuple(scratch))
  return f(table, src2d, dst2d)


def _dot_t(a, w):
  return lax.dot_general(a, w, (((1,), (1,)), ((), ())),
                         preferred_element_type=jnp.float32)


def _tc_prep(x, w_l, w_r, b, blk):
  n = x.shape[0]

  def body(x_ref, wl_ref, wr_ref, b_ref, xl_ref, xr_ref):
    xv = x_ref[...]
    xl_ref[...] = _dot_t(xv, wl_ref[...])
    xr_ref[...] = _dot_t(xv, wr_ref[...]) + b_ref[...]

  return pl.pallas_call(
      body,
      grid=(n // blk,),
      in_specs=[
          pl.BlockSpec((blk, 128), lambda i: (i, 0)),
          pl.BlockSpec((128, 128), lambda i: (0, 0)),
          pl.BlockSpec((128, 128), lambda i: (0, 0)),
          pl.BlockSpec((1, 128), lambda i: (0, 0)),
      ],
      out_specs=[
          pl.BlockSpec((blk, 128), lambda i: (i, 0)),
          pl.BlockSpec((blk, 128), lambda i: (i, 0)),
      ],
      out_shape=[
          jax.ShapeDtypeStruct((n, 128), jnp.float32),
          jax.ShapeDtypeStruct((n, 128), jnp.float32),
      ],
  )(x, w_l, w_r, b)


def _tc_mid(aggp, cntp, xr, w_l, w_r, b, blk):
  n = xr.shape[0]

  def body(p_ref, c_ref, xr_ref, wl_ref, wr_ref, b_ref, hl_ref, hr_ref):
    a = p_ref[0] + p_ref[1]
    cnt = c_ref[0, :, 0:1] + c_ref[1, :, 0:1]
    inv = 1.0 / jnp.maximum(cnt, 1.0)
    h = jnp.maximum(a * inv + xr_ref[...], 0.0)
    hl_ref[...] = _dot_t(h, wl_ref[...])
    hr_ref[...] = _dot_t(h, wr_ref[...]) + b_ref[...]

  return pl.pallas_call(
      body,
      grid=(n // blk,),
      in_specs=[
          pl.BlockSpec((NC, blk, 128), lambda i: (0, i, 0)),
          pl.BlockSpec((NC, blk, LANES), lambda i: (0, i, 0)),
          pl.BlockSpec((blk, 128), lambda i: (i, 0)),
          pl.BlockSpec((128, 128), lambda i: (0, 0)),
          pl.BlockSpec((128, 128), lambda i: (0, 0)),
          pl.BlockSpec((1, 128), lambda i: (0, 0)),
      ],
      out_specs=[
          pl.BlockSpec((blk, 128), lambda i: (i, 0)),
          pl.BlockSpec((blk, 128), lambda i: (i, 0)),
      ],
      out_shape=[
          jax.ShapeDtypeStruct((n, 128), jnp.float32),
          jax.ShapeDtypeStruct((n, 128), jnp.float32),
      ],
  )(aggp, cntp, xr, w_l, w_r, b)


def _tc_final(aggp, cntp, hr, blk):
  n = hr.shape[0]

  def body(p_ref, c_ref, hr_ref, o_ref):
    a = p_ref[0] + p_ref[1]
    cnt = c_ref[0, :, 0:1] + c_ref[1, :, 0:1]
    inv = 1.0 / jnp.maximum(cnt, 1.0)
    o_ref[...] = a * inv + hr_ref[...]

  return pl.pallas_call(
      body,
      grid=(n // blk,),
      in_specs=[
          pl.BlockSpec((NC, blk, 128), lambda i: (0, i, 0)),
          pl.BlockSpec((NC, blk, LANES), lambda i: (0, i, 0)),
          pl.BlockSpec((blk, 128), lambda i: (i, 0)),
      ],
      out_specs=pl.BlockSpec((blk, 128), lambda i: (i, 0)),
      out_shape=jax.ShapeDtypeStruct((n, 128), jnp.float32),
  )(aggp, cntp, hr)


@jax.jit
def kernel(x, edge_index, W1_l, b1, W1_r, W2_l, b2, W2_r):
  n = x.shape[0]
  e = edge_index.shape[1]

  npad = -((n + 1) // -(NS * ZROWS)) * (NS * ZROWS)
  k = -(e // -(NW * C))
  k += k % 2
  e_pad = NW * C * k

  src = edge_index[0]
  dst = edge_index[1]
  pad_src = jnp.zeros((e_pad - e,), jnp.int32)
  pad_dst = jnp.full((e_pad - e,), npad - 1, jnp.int32)
  src2d = jnp.concatenate([src, pad_src]).reshape(NW * k, C)
  dst2d = jnp.concatenate([dst, pad_dst]).reshape(NW * k, C)

  blk = 2000 if n % 2000 == 0 else (8 * (n // 8) if n % 8 else n)
  b1r = b1.reshape(1, 128)
  b2r = b2.reshape(1, 128)

  xl, xr = _tc_prep(x, W1_l, W1_r, b1r, blk)
  aggp, cntp = _sc_segment_sum(xl, src2d, dst2d, npad, with_cnt=True)
  hl, hr = _tc_mid(aggp, cntp, xr, W2_l, W2_r, b2r, blk)
  agg2p, = _sc_segment_sum(hl, src2d, dst2d, npad, with_cnt=False)
  return _tc_final(agg2p, cntp, hr, blk)


# speedup vs baseline: 3.3720x; 3.3720x over previous
import jax
import jax.numpy as jnp
from jax import lax
from jax.experimental import pallas as pl
from jax.experimental.pallas import tpu as pltpu
from jax.experimental.pallas import tpu_sc as plsc

NC = 2
NS = 16
LANES = 16
NW = NC * NS
C = 64
SI = 32
ZROWS = 64


def _mesh():
  return plsc.VectorSubcoreMesh(
      core_axis_name="c", subcore_axis_name="s",
      num_cores=NC, num_subcores=NS)


def _sc_segment_sum(table, src2d, dst2d, zeros_hbm, npad):
  n_chunks = src2d.shape[0]
  k = n_chunks // NW
  nstages = k // SI
  rows = npad // NS

  def body(table_h, src_h, dst_h, zeros_h, agg_h,
           idx_s, idx_d, buf0, buf1, acc, sem0, sem1):
    cid = lax.axis_index("c")
    sid = lax.axis_index("s")
    wid = cid * NS + sid

    base = sid * rows
    pltpu.sync_copy(zeros_h, acc.at[pl.ds(base, rows)])

    plsc.subcore_barrier()

    def do_chunk(j, buf, sem):
      pltpu.make_async_copy(table_h.at[idx_s.at[j]], buf, sem).wait()
      pltpu.sync_copy(buf, acc.at[idx_d.at[j]], add=True)

      @pl.when(j + 2 < SI)
      def _():
        pltpu.async_copy(table_h.at[idx_s.at[j + 2]], buf, sem)

    for st in range(nstages):
      row0 = wid * k + st * SI
      pltpu.sync_copy(src_h.at[pl.ds(row0, SI)], idx_s)
      pltpu.sync_copy(dst_h.at[pl.ds(row0, SI)], idx_d)
      pltpu.async_copy(table_h.at[idx_s.at[0]], buf0, sem0)
      pltpu.async_copy(table_h.at[idx_s.at[1]], buf1, sem1)

      def loop_body(i2, _):
        do_chunk(i2 * 2, buf0, sem0)
        do_chunk(i2 * 2 + 1, buf1, sem1)
        return 0
      lax.fori_loop(0, SI // 2, loop_body, 0)

    plsc.subcore_barrier()

    pltpu.sync_copy(acc.at[pl.ds(base, rows)],
                    agg_h.at[cid, pl.ds(base, rows)])

  f = pl.kernel(
      body,
      out_type=jax.ShapeDtypeStruct((NC, npad, 128), jnp.float32),
      mesh=_mesh(),
      scratch_types=(
          pltpu.VMEM((SI, C), jnp.int32),
          pltpu.VMEM((SI, C), jnp.int32),
          pltpu.VMEM((C, 128), jnp.float32),
          pltpu.VMEM((C, 128), jnp.float32),
          pltpu.VMEM_SHARED((npad, 128), jnp.float32),
          pltpu.SemaphoreType.DMA,
          pltpu.SemaphoreType.DMA,
      ))
  return f(table, src2d, dst2d, zeros_hbm)


def _sc_degree(dst2d, ones_hbm, zeros_hbm, npad):
  n_chunks = dst2d.shape[0]
  k = n_chunks // NW
  rows = npad // NS

  def body(dst_h, ones_h, zeros_h, cnt_h, idx_d, ones, cacc):
    cid = lax.axis_index("c")
    sid = lax.axis_index("s")
    wid = cid * NS + sid

    pltpu.sync_copy(dst_h.at[pl.ds(wid * k, k)], idx_d)
    pltpu.sync_copy(ones_h, ones)

    base = sid * rows
    pltpu.sync_copy(zeros_h, cacc.at[pl.ds(base, rows)])

    plsc.subcore_barrier()

    def loop_body(j, _):
      pltpu.sync_copy(ones, cacc.at[idx_d.at[j]], add=True)
      return 0
    lax.fori_loop(0, k, loop_body, 0)

    plsc.subcore_barrier()

    pltpu.sync_copy(cacc.at[pl.ds(base, rows)],
                    cnt_h.at[cid, pl.ds(base, rows)])

  f = pl.kernel(
      body,
      out_type=jax.ShapeDtypeStruct((NC, npad, 128), jnp.float32),
      mesh=_mesh(),
      scratch_types=(
          pltpu.VMEM((k, C), jnp.int32),
          pltpu.VMEM((C, 128), jnp.float32),
          pltpu.VMEM_SHARED((npad, 128), jnp.float32),
      ))
  return f(dst2d, ones_hbm, zeros_hbm)


def _dot_t(a, w):
  return lax.dot_general(a, w, (((1,), (1,)), ((), ())),
                         preferred_element_type=jnp.float32)


def _tc_prep(x, w_l, w_r, b, blk):
  n = x.shape[0]

  def body(x_ref, wl_ref, wr_ref, b_ref, xl_ref, xr_ref):
    xv = x_ref[...]
    xl_ref[...] = _dot_t(xv, wl_ref[...])
    xr_ref[...] = _dot_t(xv, wr_ref[...]) + b_ref[...]

  return pl.pallas_call(
      body,
      grid=(n // blk,),
      in_specs=[
          pl.BlockSpec((blk, 128), lambda i: (i, 0)),
          pl.BlockSpec((128, 128), lambda i: (0, 0)),
          pl.BlockSpec((128, 128), lambda i: (0, 0)),
          pl.BlockSpec((1, 128), lambda i: (0, 0)),
      ],
      out_specs=[
          pl.BlockSpec((blk, 128), lambda i: (i, 0)),
          pl.BlockSpec((blk, 128), lambda i: (i, 0)),
      ],
      out_shape=[
          jax.ShapeDtypeStruct((n, 128), jnp.float32),
          jax.ShapeDtypeStruct((n, 128), jnp.float32),
      ],
  )(x, w_l, w_r, b)


def _tc_mid(aggp, cntp, xr, w_l, w_r, b, blk):
  n = xr.shape[0]

  def body(p_ref, c_ref, xr_ref, wl_ref, wr_ref, b_ref, hl_ref, hr_ref):
    a = p_ref[0] + p_ref[1]
    cnt = c_ref[0, :, 0:1] + c_ref[1, :, 0:1]
    inv = 1.0 / jnp.maximum(cnt, 1.0)
    h = jnp.maximum(a * inv + xr_ref[...], 0.0)
    hl_ref[...] = _dot_t(h, wl_ref[...])
    hr_ref[...] = _dot_t(h, wr_ref[...]) + b_ref[...]

  return pl.pallas_call(
      body,
      grid=(n // blk,),
      in_specs=[
          pl.BlockSpec((NC, blk, 128), lambda i: (0, i, 0)),
          pl.BlockSpec((NC, blk, 128), lambda i: (0, i, 0)),
          pl.BlockSpec((blk, 128), lambda i: (i, 0)),
          pl.BlockSpec((128, 128), lambda i: (0, 0)),
          pl.BlockSpec((128, 128), lambda i: (0, 0)),
          pl.BlockSpec((1, 128), lambda i: (0, 0)),
      ],
      out_specs=[
          pl.BlockSpec((blk, 128), lambda i: (i, 0)),
          pl.BlockSpec((blk, 128), lambda i: (i, 0)),
      ],
      out_shape=[
          jax.ShapeDtypeStruct((n, 128), jnp.float32),
          jax.ShapeDtypeStruct((n, 128), jnp.float32),
      ],
  )(aggp, cntp, xr, w_l, w_r, b)


def _tc_final(aggp, cntp, hr, blk):
  n = hr.shape[0]

  def body(p_ref, c_ref, hr_ref, o_ref):
    a = p_ref[0] + p_ref[1]
    cnt = c_ref[0, :, 0:1] + c_ref[1, :, 0:1]
    inv = 1.0 / jnp.maximum(cnt, 1.0)
    o_ref[...] = a * inv + hr_ref[...]

  return pl.pallas_call(
      body,
      grid=(n // blk,),
      in_specs=[
          pl.BlockSpec((NC, blk, 128), lambda i: (0, i, 0)),
          pl.BlockSpec((NC, blk, 128), lambda i: (0, i, 0)),
          pl.BlockSpec((blk, 128), lambda i: (i, 0)),
      ],
      out_specs=pl.BlockSpec((blk, 128), lambda i: (i, 0)),
      out_shape=jax.ShapeDtypeStruct((n, 128), jnp.float32),
  )(aggp, cntp, hr)


@jax.jit
def kernel(x, edge_index, W1_l, b1, W1_r, W2_l, b2, W2_r):
  n = x.shape[0]
  e = edge_index.shape[1]

  npad = -((n + 1) // -(NS * ZROWS)) * (NS * ZROWS)
  k = -(e // -(NW * C * SI)) * SI
  e_pad = NW * C * k

  src = edge_index[0]
  dst = edge_index[1]
  pad_src = jnp.zeros((e_pad - e,), jnp.int32)
  pad_dst = jnp.full((e_pad - e,), npad - 1, jnp.int32)
  src2d = jnp.concatenate([src, pad_src]).reshape(NW * k, C)
  dst2d = jnp.concatenate([dst, pad_dst]).reshape(NW * k, C)

  blk = 2000 if n % 2000 == 0 else (8 * (n // 8) if n % 8 else n)
  b1r = b1.reshape(1, 128)
  b2r = b2.reshape(1, 128)

  rows = npad // NS
  zeros128 = jnp.zeros((rows, 128), jnp.float32)
  ones_pat = jnp.ones((C, 128), jnp.float32)

  cntp = _sc_degree(dst2d, ones_pat, zeros128, npad)
  xl, xr = _tc_prep(x, W1_l, W1_r, b1r, blk)
  aggp = _sc_segment_sum(xl, src2d, dst2d, zeros128, npad)
  hl, hr = _tc_mid(aggp, cntp, xr, W2_l, W2_r, b2r, blk)
  agg2p = _sc_segment_sum(hl, src2d, dst2d, zeros128, npad)
  return _tc_final(agg2p, cntp, hr, blk)


# speedup vs baseline: 3.6263x; 1.0754x over previous
import jax
import jax.numpy as jnp
from jax import lax
from jax.experimental import pallas as pl
from jax.experimental.pallas import tpu as pltpu
from jax.experimental.pallas import tpu_sc as plsc

NC = 2
NS = 16
LANES = 16
NW = NC * NS
C = 128
SI = 16
ZROWS = 64


def _mesh():
  return plsc.VectorSubcoreMesh(
      core_axis_name="c", subcore_axis_name="s",
      num_cores=NC, num_subcores=NS)


def _sc_segment_sum(table, src2d, dst2d, zeros_hbm, npad):
  n_chunks = src2d.shape[0]
  k = n_chunks // NW
  nstages = k // SI
  rows = npad // NS

  def body(table_h, src_h, dst_h, zeros_h, agg_h,
           idx_s, idx_d, buf0, buf1, acc, sem0, sem1):
    cid = lax.axis_index("c")
    sid = lax.axis_index("s")
    wid = cid * NS + sid

    base = sid * rows
    pltpu.sync_copy(zeros_h, acc.at[pl.ds(base, rows)])

    plsc.subcore_barrier()

    def do_chunk(j, buf, sem):
      pltpu.make_async_copy(table_h.at[idx_s.at[j]], buf, sem).wait()
      pltpu.sync_copy(buf, acc.at[idx_d.at[j]], add=True)

      @pl.when(j + 2 < SI)
      def _():
        pltpu.async_copy(table_h.at[idx_s.at[j + 2]], buf, sem)

    for st in range(nstages):
      row0 = wid * k + st * SI
      pltpu.sync_copy(src_h.at[pl.ds(row0, SI)], idx_s)
      pltpu.sync_copy(dst_h.at[pl.ds(row0, SI)], idx_d)
      pltpu.async_copy(table_h.at[idx_s.at[0]], buf0, sem0)
      pltpu.async_copy(table_h.at[idx_s.at[1]], buf1, sem1)

      def loop_body(i2, _):
        do_chunk(i2 * 2, buf0, sem0)
        do_chunk(i2 * 2 + 1, buf1, sem1)
        return 0
      lax.fori_loop(0, SI // 2, loop_body, 0)

    plsc.subcore_barrier()

    pltpu.sync_copy(acc.at[pl.ds(base, rows)],
                    agg_h.at[cid, pl.ds(base, rows)])

  f = pl.kernel(
      body,
      out_type=jax.ShapeDtypeStruct((NC, npad, 128), jnp.float32),
      mesh=_mesh(),
      scratch_types=(
          pltpu.VMEM((SI, C), jnp.int32),
          pltpu.VMEM((SI, C), jnp.int32),
          pltpu.VMEM((C, 128), jnp.float32),
          pltpu.VMEM((C, 128), jnp.float32),
          pltpu.VMEM_SHARED((npad, 128), jnp.float32),
          pltpu.SemaphoreType.DMA,
          pltpu.SemaphoreType.DMA,
      ))
  return f(table, src2d, dst2d, zeros_hbm)


def _sc_degree(dst2d, ones_hbm, zeros_hbm, npad):
  n_chunks = dst2d.shape[0]
  k = n_chunks // NW
  rows = npad // NS

  def body(dst_h, ones_h, zeros_h, cnt_h, idx_d, ones, cacc):
    cid = lax.axis_index("c")
    sid = lax.axis_index("s")
    wid = cid * NS + sid

    pltpu.sync_copy(dst_h.at[pl.ds(wid * k, k)], idx_d)
    pltpu.sync_copy(ones_h, ones)

    base = sid * rows
    pltpu.sync_copy(zeros_h, cacc.at[pl.ds(base, rows)])

    plsc.subcore_barrier()

    def loop_body(j, _):
      pltpu.sync_copy(ones, cacc.at[idx_d.at[j]], add=True)
      return 0
    lax.fori_loop(0, k, loop_body, 0)

    plsc.subcore_barrier()

    pltpu.sync_copy(cacc.at[pl.ds(base, rows)],
                    cnt_h.at[cid, pl.ds(base, rows)])

  f = pl.kernel(
      body,
      out_type=jax.ShapeDtypeStruct((NC, npad, 128), jnp.float32),
      mesh=_mesh(),
      scratch_types=(
          pltpu.VMEM((k, C), jnp.int32),
          pltpu.VMEM((C, 128), jnp.float32),
          pltpu.VMEM_SHARED((npad, 128), jnp.float32),
      ))
  return f(dst2d, ones_hbm, zeros_hbm)


def _dot_t(a, w):
  return lax.dot_general(a, w, (((1,), (1,)), ((), ())),
                         preferred_element_type=jnp.float32)


def _tc_prep(x, w_l, w_r, b, blk):
  n = x.shape[0]

  def body(x_ref, wl_ref, wr_ref, b_ref, xl_ref, xr_ref):
    xv = x_ref[...]
    xl_ref[...] = _dot_t(xv, wl_ref[...])
    xr_ref[...] = _dot_t(xv, wr_ref[...]) + b_ref[...]

  return pl.pallas_call(
      body,
      grid=(n // blk,),
      in_specs=[
          pl.BlockSpec((blk, 128), lambda i: (i, 0)),
          pl.BlockSpec((128, 128), lambda i: (0, 0)),
          pl.BlockSpec((128, 128), lambda i: (0, 0)),
          pl.BlockSpec((1, 128), lambda i: (0, 0)),
      ],
      out_specs=[
          pl.BlockSpec((blk, 128), lambda i: (i, 0)),
          pl.BlockSpec((blk, 128), lambda i: (i, 0)),
      ],
      out_shape=[
          jax.ShapeDtypeStruct((n, 128), jnp.float32),
          jax.ShapeDtypeStruct((n, 128), jnp.float32),
      ],
  )(x, w_l, w_r, b)


def _tc_mid(aggp, cntp, xr, w_l, w_r, b, blk):
  n = xr.shape[0]

  def body(p_ref, c_ref, xr_ref, wl_ref, wr_ref, b_ref, hl_ref, hr_ref):
    a = p_ref[0] + p_ref[1]
    cnt = c_ref[0, :, 0:1] + c_ref[1, :, 0:1]
    inv = 1.0 / jnp.maximum(cnt, 1.0)
    h = jnp.maximum(a * inv + xr_ref[...], 0.0)
    hl_ref[...] = _dot_t(h, wl_ref[...])
    hr_ref[...] = _dot_t(h, wr_ref[...]) + b_ref[...]

  return pl.pallas_call(
      body,
      grid=(n // blk,),
      in_specs=[
          pl.BlockSpec((NC, blk, 128), lambda i: (0, i, 0)),
          pl.BlockSpec((NC, blk, 128), lambda i: (0, i, 0)),
          pl.BlockSpec((blk, 128), lambda i: (i, 0)),
          pl.BlockSpec((128, 128), lambda i: (0, 0)),
          pl.BlockSpec((128, 128), lambda i: (0, 0)),
          pl.BlockSpec((1, 128), lambda i: (0, 0)),
      ],
      out_specs=[
          pl.BlockSpec((blk, 128), lambda i: (i, 0)),
          pl.BlockSpec((blk, 128), lambda i: (i, 0)),
      ],
      out_shape=[
          jax.ShapeDtypeStruct((n, 128), jnp.float32),
          jax.ShapeDtypeStruct((n, 128), jnp.float32),
      ],
  )(aggp, cntp, xr, w_l, w_r, b)


def _tc_final(aggp, cntp, hr, blk):
  n = hr.shape[0]

  def body(p_ref, c_ref, hr_ref, o_ref):
    a = p_ref[0] + p_ref[1]
    cnt = c_ref[0, :, 0:1] + c_ref[1, :, 0:1]
    inv = 1.0 / jnp.maximum(cnt, 1.0)
    o_ref[...] = a * inv + hr_ref[...]

  return pl.pallas_call(
      body,
      grid=(n // blk,),
      in_specs=[
          pl.BlockSpec((NC, blk, 128), lambda i: (0, i, 0)),
          pl.BlockSpec((NC, blk, 128), lambda i: (0, i, 0)),
          pl.BlockSpec((blk, 128), lambda i: (i, 0)),
      ],
      out_specs=pl.BlockSpec((blk, 128), lambda i: (i, 0)),
      out_shape=jax.ShapeDtypeStruct((n, 128), jnp.float32),
  )(aggp, cntp, hr)


@jax.jit
def kernel(x, edge_index, W1_l, b1, W1_r, W2_l, b2, W2_r):
  n = x.shape[0]
  e = edge_index.shape[1]

  npad = -((n + 1) // -(NS * ZROWS)) * (NS * ZROWS)
  k = -(e // -(NW * C * SI)) * SI
  e_pad = NW * C * k

  src = edge_index[0]
  dst = edge_index[1]
  pad_src = jnp.zeros((e_pad - e,), jnp.int32)
  pad_dst = jnp.full((e_pad - e,), npad - 1, jnp.int32)
  src2d = jnp.concatenate([src, pad_src]).reshape(NW * k, C)
  dst2d = jnp.concatenate([dst, pad_dst]).reshape(NW * k, C)

  blk = 2000 if n % 2000 == 0 else (8 * (n // 8) if n % 8 else n)
  b1r = b1.reshape(1, 128)
  b2r = b2.reshape(1, 128)

  rows = npad // NS
  zeros128 = jnp.zeros((rows, 128), jnp.float32)
  ones_pat = jnp.ones((C, 128), jnp.float32)

  cntp = _sc_degree(dst2d, ones_pat, zeros128, npad)
  xl, xr = _tc_prep(x, W1_l, W1_r, b1r, blk)
  aggp = _sc_segment_sum(xl, src2d, dst2d, zeros128, npad)
  hl, hr = _tc_mid(aggp, cntp, xr, W2_l, W2_r, b2r, blk)
  agg2p = _sc_segment_sum(hl, src2d, dst2d, zeros128, npad)
  return _tc_final(agg2p, cntp, hr, blk)
